# baseline (speedup 1.0000x reference)
import functools

import jax
import jax.numpy as jnp
from jax import lax
from jax.experimental import pallas as pl
from jax.experimental.pallas import tpu as pltpu
from jax.experimental.pallas import tpu_sc as plsc

B, N, D, E = 2, 2048, 1024, 8
TOP_K = 2
NUM_EXPERTS_B = 4
FLOOR = min(0.05, 0.15 / max(NUM_EXPERTS_B, 1))
ALPHA = min(FLOOR * E, 1.0)
CAP_LOW, CAP_HIGH, T_MAX = 0.5, 0.6, 1000
NT = B * N
H = D // 2
K_SEL = min(max(1, N * TOP_K // E), N)
BT = 512
NBLK = NT // BT
INT_MIN = -2147483648
NROW = B * E
L = 16
NCH = N // L


def _logits_body(tok_ref, xyzT_ref, xyzTb_ref, W1at_ref, W1bt_ref, b1_ref,
                 W2t_ref, b2_ref, centers_ref, logitsT_ref,
                 distsT_s, eye_s, inv_s):
    i = pl.program_id(0)

    @pl.when(i == 0)
    def _prologue():
        r = lax.broadcasted_iota(jnp.int32, (BT, BT), 0)
        c = lax.broadcasted_iota(jnp.int32, (BT, BT), 1)
        eye_s[...] = (r == c).astype(jnp.float32)
        d2 = jnp.zeros((E, NT), jnp.float32)
        for j in range(3):
            xr = xyzT_ref[j:j + 1, :]
            cc = centers_ref[:, j:j + 1]
            diff = cc - xr
            d2 = d2 + diff * diff
        dTmat = jnp.sqrt(d2)
        distsT_s[...] = dTmat
        mean = jnp.sum(dTmat) / (B * N * E)
        inv_s[0, 0] = 1.0 / (mean + 1e-6)

    inv = inv_s[0, 0]
    xslice = xyzTb_ref[:, pl.ds(i * BT, BT)]
    pre = jnp.dot(tok_ref[...], W1at_ref[...],
                  preferred_element_type=jnp.float32)
    pre = pre + lax.dot_general(xslice, W1bt_ref[...],
                                (((0,), (0,)), ((), ())),
                                preferred_element_type=jnp.float32)
    pre = pre + b1_ref[...]
    h = 0.5 * pre * (1.0 + lax.erf(pre * (2.0 ** -0.5)))
    content = jnp.dot(h.astype(jnp.bfloat16), W2t_ref[...],
                      preferred_element_type=jnp.float32) + b2_ref[...]
    contentT = lax.dot_general(content, eye_s[...],
                               (((0,), (0,)), ((), ())),
                               precision=lax.Precision.HIGHEST,
                               preferred_element_type=jnp.float32)
    aff = distsT_s[:, pl.ds(i * BT, BT)] * (-inv)
    lT = contentT + aff
    b = i // (N // BT)
    off = (i % (N // BT)) * BT
    logitsT_ref[pl.ds(b * E, E), pl.ds(off, BT)] = lT


def _sc_select_body(logitsT_hbm, out_hbm, row_v, keys_v, disp_v):
    wid = lax.axis_index("s") * 2 + lax.axis_index("c")

    @pl.when(wid < NROW)
    def _():
        pltpu.sync_copy(logitsT_hbm.at[wid], row_v)

        @plsc.parallel_loop(0, NCH, unroll=8)
        def _keys(j):
            x = row_v[pl.ds(j * L, L)]
            bits = lax.bitcast_convert_type(x, jnp.int32)
            key = jnp.where(bits < 0, jnp.int32(INT_MIN) - bits, bits)
            keys_v[pl.ds(j * L, L)] = key

        def count_ge(candv):
            def cbody(j, acc):
                kc = keys_v[pl.ds(j * L, L)]
                return acc + plsc.all_reduce_population_count(kc >= candv)
            return plsc.parallel_loop(
                0, NCH, unroll=8,
                carry=jnp.zeros((L,), jnp.int32))(cbody)

        kkv = jnp.full((L,), K_SEL, jnp.int32)
        zerov = jnp.zeros((L,), jnp.int32)
        minv = jnp.full((L,), INT_MIN, jnp.int32)
        T0 = jnp.where(count_ge(zerov) >= kkv, zerov, minv)

        def bs(it, T):
            bit = 30 - it
            cand = T | (zerov + lax.shift_left(jnp.int32(1), bit))
            return jnp.where(count_ge(cand) >= kkv, cand, T)

        T = lax.fori_loop(0, 31, bs, T0)

        def count_gt(candv):
            def cbody(j, acc):
                kc = keys_v[pl.ds(j * L, L)]
                return acc + plsc.all_reduce_population_count(kc > candv)
            return plsc.parallel_loop(
                0, NCH, unroll=8,
                carry=jnp.zeros((L,), jnp.int32))(cbody)

        rr = kkv - count_gt(T)

        def count_eq_lt(Iv):
            def cbody(j, acc):
                kc = keys_v[pl.ds(j * L, L)]
                idxc = lax.iota(jnp.int32, L) + j * L
                m = (kc == T) & (idxc < Iv)
                return acc + plsc.all_reduce_population_count(m)
            return plsc.parallel_loop(
                0, NCH, unroll=8,
                carry=jnp.zeros((L,), jnp.int32))(cbody)

        nv = jnp.full((L,), N, jnp.int32)

        def jpb(it, J):
            bit = 11 - it
            cand = J | (zerov + lax.shift_left(jnp.int32(1), bit))
            ok = (cand <= nv) & (count_eq_lt(cand) < rr)
            return jnp.where(ok, cand, J)

        Jp = lax.fori_loop(0, 12, jpb, zerov)

        @plsc.parallel_loop(0, NCH, unroll=8)
        def _fill(j):
            x = row_v[pl.ds(j * L, L)]
            kc = keys_v[pl.ds(j * L, L)]
            idxc = lax.iota(jnp.int32, L) + j * L
            sel = (kc > T) | ((kc == T) & (idxc <= Jp))
            sig = 1.0 / (1.0 + jnp.exp(-x))
            d0 = jnp.where(sel, sig, 0.0)
            disp_v[pl.ds(j * L, L)] = (1.0 - ALPHA) * d0 + (ALPHA / E)
        pltpu.sync_copy(disp_v, out_hbm.at[wid])


def _sc_select(logitsT):
    mesh = plsc.VectorSubcoreMesh(core_axis_name="c", subcore_axis_name="s")
    fn = functools.partial(
        pl.kernel,
        mesh=mesh,
        compiler_params=pltpu.CompilerParams(needs_layout_passes=False),
        out_type=jax.ShapeDtypeStruct((NROW, N), jnp.float32),
        scratch_types=[
            pltpu.VMEM((N,), jnp.float32),
            pltpu.VMEM((N,), jnp.int32),
            pltpu.VMEM((N,), jnp.float32),
        ],
    )(_sc_select_body)
    return fn(logitsT)


def _finish_body(d1_ref, t_ref, disp_ref, comb_ref):
    d1 = d1_ref[...]
    t0 = t_ref[0].astype(jnp.float32)
    t1 = t_ref[1].astype(jnp.float32)
    cap0 = CAP_LOW + (CAP_HIGH + CAP_LOW) * (t0 / T_MAX)
    cap1 = CAP_LOW + (CAP_HIGH + CAP_LOW) * (t1 / T_MAX)
    riota = lax.broadcasted_iota(jnp.int32, (NROW, 1), 0)
    cap = jnp.where(riota < E, cap0, cap1)

    def bsum(X):
        s0 = jnp.sum(X[0:E, :], axis=0, keepdims=True)
        s1 = jnp.sum(X[E:2 * E, :], axis=0, keepdims=True)
        return jnp.concatenate([jnp.broadcast_to(s0, (E, N)),
                                jnp.broadcast_to(s1, (E, N))], axis=0)

    excess = jnp.maximum(d1 - cap, 0.0)
    capped = d1 - excess
    headroom = jnp.maximum(cap - capped, 0.0)
    hsum = jnp.maximum(bsum(headroom), 1e-8)
    capped = capped + bsum(excess) * (headroom / hsum)
    comb = capped / (bsum(capped) + 1e-8)
    r8 = lax.broadcasted_iota(jnp.int32, (E, E), 0)
    c8 = lax.broadcasted_iota(jnp.int32, (E, E), 1)
    eye8 = (r8 == c8).astype(jnp.float32)
    for b in range(B):
        for c4 in range(N // BT):
            for src, dst in ((capped, disp_ref), (comb, comb_ref)):
                tile = src[b * E:(b + 1) * E, c4 * BT:(c4 + 1) * BT]
                tileT = lax.dot_general(tile, eye8, (((0,), (0,)), ((), ())),
                                        precision=lax.Precision.HIGHEST,
                                        preferred_element_type=jnp.float32)
                dst[pl.ds(b * N + c4 * BT, BT), :] = tileT


@jax.jit
def kernel(tokens, spatial_xyz, W1, b1, W2, b2, centers, t):
    tok = tokens.reshape(NT, D).astype(jnp.bfloat16)
    xyzT = spatial_xyz.reshape(NT, 3).T
    xyzTb = xyzT.astype(jnp.bfloat16)
    W1at = W1[:, :D].T.astype(jnp.bfloat16)
    W1bt = W1[:, D:].T.astype(jnp.bfloat16)
    W2t = W2.T.astype(jnp.bfloat16)
    b1r = b1.reshape(1, H)
    b2r = b2.reshape(1, E)

    logitsT = pl.pallas_call(
        _logits_body,
        grid=(NBLK,),
        in_specs=[
            pl.BlockSpec((BT, D), lambda i: (i, 0)),
            pl.BlockSpec((3, NT), lambda i: (0, 0)),
            pl.BlockSpec((3, NT), lambda i: (0, 0)),
            pl.BlockSpec((D, H), lambda i: (0, 0)),
            pl.BlockSpec((3, H), lambda i: (0, 0)),
            pl.BlockSpec((1, H), lambda i: (0, 0)),
            pl.BlockSpec((H, E), lambda i: (0, 0)),
            pl.BlockSpec((1, E), lambda i: (0, 0)),
            pl.BlockSpec((E, 3), lambda i: (0, 0)),
        ],
        out_specs=pl.BlockSpec((NROW, N), lambda i: (0, 0)),
        out_shape=jax.ShapeDtypeStruct((NROW, N), jnp.float32),
        scratch_shapes=[
            pltpu.VMEM((E, NT), jnp.float32),
            pltpu.VMEM((BT, BT), jnp.float32),
            pltpu.SMEM((1, 1), jnp.float32),
        ],
    )(tok, xyzT, xyzTb, W1at, W1bt, b1r, W2t, b2r, centers)

    d0 = _sc_select(logitsT)

    dT, cT = pl.pallas_call(
        _finish_body,
        grid=(1,),
        in_specs=[
            pl.BlockSpec((NROW, N), lambda i: (0, 0)),
            pl.BlockSpec(memory_space=pltpu.SMEM),
        ],
        out_specs=[
            pl.BlockSpec((NT, E), lambda i: (0, 0)),
            pl.BlockSpec((NT, E), lambda i: (0, 0)),
        ],
        out_shape=[
            jax.ShapeDtypeStruct((NT, E), jnp.float32),
            jax.ShapeDtypeStruct((NT, E), jnp.float32),
        ],
    )(d0, t)

    dispatch = dT.reshape(B, N, E)
    combine = cT.reshape(B, N, E)
    return (dispatch, combine)


# speedup vs baseline: 1.3392x; 1.3392x over previous
import functools

import jax
import jax.numpy as jnp
from jax import lax
from jax.experimental import pallas as pl
from jax.experimental.pallas import tpu as pltpu
from jax.experimental.pallas import tpu_sc as plsc

B, N, D, E = 2, 2048, 1024, 8
TOP_K = 2
NUM_EXPERTS_B = 4
FLOOR = min(0.05, 0.15 / max(NUM_EXPERTS_B, 1))
ALPHA = min(FLOOR * E, 1.0)
CAP_LOW, CAP_HIGH, T_MAX = 0.5, 0.6, 1000
NT = B * N
H = D // 2
K_SEL = min(max(1, N * TOP_K // E), N)
BT = 512
NBLK = NT // BT
INT_MIN = -2147483648
NROW = B * E
L = 16
NCH = N // L


def _logits_body(tok_ref, xyzT_ref, xyzTb_ref, W1_ref, b1_ref,
                 W2_ref, b2c_ref, centers_ref, logitsT_ref,
                 distsT_s, W1b_s, inv_s):
    i = pl.program_id(0)

    @pl.when(i == 0)
    def _prologue():
        W1b_s[...] = W1_ref[...].astype(jnp.bfloat16)
        d2 = jnp.zeros((E, NT), jnp.float32)
        for j in range(3):
            xr = xyzT_ref[j:j + 1, :]
            cc = centers_ref[:, j:j + 1]
            diff = cc - xr
            d2 = d2 + diff * diff
        dTmat = jnp.sqrt(d2)
        distsT_s[...] = dTmat
        mean = jnp.sum(dTmat) / (B * N * E)
        inv_s[0, 0] = 1.0 / (mean + 1e-6)

    inv = inv_s[0, 0]
    xslice = xyzTb_ref[:, pl.ds(i * BT, BT)]
    pre = lax.dot_general(tok_ref[...].astype(jnp.bfloat16), W1b_s[:, :D],
                          (((1,), (1,)), ((), ())),
                          preferred_element_type=jnp.float32)
    pre = pre + lax.dot_general(xslice, W1b_s[:, D:],
                                (((0,), (1,)), ((), ())),
                                preferred_element_type=jnp.float32)
    pre = pre + b1_ref[...]
    h = 0.5 * pre * (1.0 + lax.erf(pre * (2.0 ** -0.5)))
    contentT = lax.dot_general(W2_ref[...].astype(jnp.bfloat16),
                               h.astype(jnp.bfloat16),
                               (((1,), (1,)), ((), ())),
                               preferred_element_type=jnp.float32)
    contentT = contentT + b2c_ref[...]
    aff = distsT_s[:, pl.ds(i * BT, BT)] * (-inv)
    lT = contentT + aff
    b = i // (N // BT)
    off = (i % (N // BT)) * BT
    logitsT_ref[pl.ds(b * E, E), pl.ds(off, BT)] = lT


def _sc_select_body(logitsT_hbm, out_hbm, row_v, keys_v, disp_v):
    wid = lax.axis_index("s") * 2 + lax.axis_index("c")

    @pl.when(wid < NROW)
    def _():
        pltpu.sync_copy(logitsT_hbm.at[wid], row_v)

        @plsc.parallel_loop(0, NCH, unroll=8)
        def _keys(j):
            x = row_v[pl.ds(j * L, L)]
            bits = lax.bitcast_convert_type(x, jnp.int32)
            key = jnp.where(bits < 0, jnp.int32(INT_MIN) - bits, bits)
            keys_v[pl.ds(j * L, L)] = key

        def count_ge(candv):
            def cbody(j, acc):
                kc = keys_v[pl.ds(j * L, L)]
                return acc + plsc.all_reduce_population_count(kc >= candv)
            return plsc.parallel_loop(
                0, NCH, unroll=8,
                carry=jnp.zeros((L,), jnp.int32))(cbody)

        kkv = jnp.full((L,), K_SEL, jnp.int32)
        zerov = jnp.zeros((L,), jnp.int32)
        minv = jnp.full((L,), INT_MIN, jnp.int32)
        T0 = jnp.where(count_ge(zerov) >= kkv, zerov, minv)

        def bs(it, T):
            bit = 30 - it
            cand = T | (zerov + lax.shift_left(jnp.int32(1), bit))
            return jnp.where(count_ge(cand) >= kkv, cand, T)

        T = lax.fori_loop(0, 31, bs, T0)

        def count_gt(candv):
            def cbody(j, acc):
                kc = keys_v[pl.ds(j * L, L)]
                return acc + plsc.all_reduce_population_count(kc > candv)
            return plsc.parallel_loop(
                0, NCH, unroll=8,
                carry=jnp.zeros((L,), jnp.int32))(cbody)

        rr = kkv - count_gt(T)

        def count_eq_lt(Iv):
            def cbody(j, acc):
                kc = keys_v[pl.ds(j * L, L)]
                idxc = lax.iota(jnp.int32, L) + j * L
                m = (kc == T) & (idxc < Iv)
                return acc + plsc.all_reduce_population_count(m)
            return plsc.parallel_loop(
                0, NCH, unroll=8,
                carry=jnp.zeros((L,), jnp.int32))(cbody)

        nv = jnp.full((L,), N, jnp.int32)

        def jpb(it, J):
            bit = 11 - it
            cand = J | (zerov + lax.shift_left(jnp.int32(1), bit))
            ok = (cand <= nv) & (count_eq_lt(cand) < rr)
            return jnp.where(ok, cand, J)

        Jp = lax.fori_loop(0, 12, jpb, zerov)

        @plsc.parallel_loop(0, NCH, unroll=8)
        def _fill(j):
            x = row_v[pl.ds(j * L, L)]
            kc = keys_v[pl.ds(j * L, L)]
            idxc = lax.iota(jnp.int32, L) + j * L
            sel = (kc > T) | ((kc == T) & (idxc <= Jp))
            sig = 1.0 / (1.0 + jnp.exp(-x))
            disp_v[pl.ds(j * L, L)] = jnp.where(sel, sig, 0.0)
        pltpu.sync_copy(disp_v, out_hbm.at[wid])


def _sc_select(logitsT):
    mesh = plsc.VectorSubcoreMesh(core_axis_name="c", subcore_axis_name="s")
    fn = functools.partial(
        pl.kernel,
        mesh=mesh,
        compiler_params=pltpu.CompilerParams(needs_layout_passes=False),
        out_type=jax.ShapeDtypeStruct((NROW, N), jnp.float32),
        scratch_types=[
            pltpu.VMEM((N,), jnp.float32),
            pltpu.VMEM((N,), jnp.int32),
            pltpu.VMEM((N,), jnp.float32),
        ],
    )(_sc_select_body)
    return fn(logitsT)


def _finish_body(d0_ref, t_ref, dT_ref, cT_ref):
    d0 = d0_ref[...]
    d1 = (1.0 - ALPHA) * d0 + (ALPHA / E)
    t0 = t_ref[0].astype(jnp.float32)
    t1 = t_ref[1].astype(jnp.float32)
    cap0 = CAP_LOW + (CAP_HIGH + CAP_LOW) * (t0 / T_MAX)
    cap1 = CAP_LOW + (CAP_HIGH + CAP_LOW) * (t1 / T_MAX)
    riota = lax.broadcasted_iota(jnp.int32, (NROW, 1), 0)
    cap = jnp.where(riota < E, cap0, cap1)

    def bsum(X):
        s0 = jnp.sum(X[0:E, :], axis=0, keepdims=True)
        s1 = jnp.sum(X[E:2 * E, :], axis=0, keepdims=True)
        return jnp.concatenate([jnp.broadcast_to(s0, (E, N)),
                                jnp.broadcast_to(s1, (E, N))], axis=0)

    excess = jnp.maximum(d1 - cap, 0.0)
    capped = d1 - excess
    headroom = jnp.maximum(cap - capped, 0.0)
    hsum = jnp.maximum(bsum(headroom), 1e-8)
    capped = capped + bsum(excess) * (headroom / hsum)
    dT_ref[...] = capped
    cT_ref[...] = capped / (bsum(capped) + 1e-8)


@jax.jit
def kernel(tokens, spatial_xyz, W1, b1, W2, b2, centers, t):
    tok = tokens.reshape(NT, D)
    xyzT = spatial_xyz.reshape(NT, 3).T
    xyzTb = xyzT.astype(jnp.bfloat16)
    b1r = b1.reshape(1, H)
    b2c = b2.reshape(E, 1)

    logitsT = pl.pallas_call(
        _logits_body,
        grid=(NBLK,),
        in_specs=[
            pl.BlockSpec((BT, D), lambda i: (i, 0)),
            pl.BlockSpec((3, NT), lambda i: (0, 0)),
            pl.BlockSpec((3, NT), lambda i: (0, 0)),
            pl.BlockSpec((H, D + 3), lambda i: (0, 0)),
            pl.BlockSpec((1, H), lambda i: (0, 0)),
            pl.BlockSpec((E, H), lambda i: (0, 0)),
            pl.BlockSpec((E, 1), lambda i: (0, 0)),
            pl.BlockSpec((E, 3), lambda i: (0, 0)),
        ],
        out_specs=pl.BlockSpec((NROW, N), lambda i: (0, 0)),
        out_shape=jax.ShapeDtypeStruct((NROW, N), jnp.float32),
        scratch_shapes=[
            pltpu.VMEM((E, NT), jnp.float32),
            pltpu.VMEM((H, D + 3), jnp.bfloat16),
            pltpu.SMEM((1, 1), jnp.float32),
        ],
    )(tok, xyzT, xyzTb, W1, b1r, W2, b2c, centers)

    d0 = _sc_select(logitsT)

    dT, cT = pl.pallas_call(
        _finish_body,
        grid=(1,),
        in_specs=[
            pl.BlockSpec((NROW, N), lambda i: (0, 0)),
            pl.BlockSpec(memory_space=pltpu.SMEM),
        ],
        out_specs=[
            pl.BlockSpec((NROW, N), lambda i: (0, 0)),
            pl.BlockSpec((NROW, N), lambda i: (0, 0)),
        ],
        out_shape=[
            jax.ShapeDtypeStruct((NROW, N), jnp.float32),
            jax.ShapeDtypeStruct((NROW, N), jnp.float32),
        ],
    )(d0, t)

    dispatch = dT.reshape(B, E, N).transpose(0, 2, 1)
    combine = cT.reshape(B, E, N).transpose(0, 2, 1)
    return (dispatch, combine)
